# Initial kernel scaffold; baseline (speedup 1.0000x reference)
import jax
import jax.numpy as jnp
from jax.experimental import pallas as pl


def kernel(white_features, black_features, stm, ft_weight, ft_bias, out_weight, out_bias):
    raise NotImplementedError("write your pallas kernel here")


# speedup vs baseline: 12.4949x; 12.4949x over previous
import functools

import jax
import jax.numpy as jnp
from jax.experimental import pallas as pl

NUM_FEATURES = 768
HIDDEN = 512
MAX_ACTIVE = 32
BB = 512


def _tc_body(wf_ref, bf_ref, stm_ref, table_ref, bias_ref, ow_ref, ob_ref, out_ref):
    feat_iota = jax.lax.broadcasted_iota(jnp.int32, (1, NUM_FEATURES), 1)
    table = table_ref[...]

    def counts_of(feat_ref):
        acc = jnp.zeros((BB, NUM_FEATURES), jnp.float32)
        for a in range(MAX_ACTIVE):
            col = feat_ref[:, a][:, None]
            acc = acc + (col == feat_iota).astype(jnp.float32)
        return acc

    bias = bias_ref[0, :][None, :]
    acc_w = jnp.dot(counts_of(wf_ref), table,
                    preferred_element_type=jnp.float32) + bias
    acc_b = jnp.dot(counts_of(bf_ref), table,
                    preferred_element_type=jnp.float32) + bias

    act_w = jnp.square(jnp.clip(acc_w, 0.0, 1.0))
    act_b = jnp.square(jnp.clip(acc_b, 0.0, 1.0))

    s = stm_ref[:, 0].astype(jnp.float32)[:, None]
    us = act_w + s * (act_b - act_w)
    them = act_b + s * (act_w - act_b)

    w_us = ow_ref[0, :HIDDEN][None, :]
    w_them = ow_ref[0, HIDDEN:][None, :]
    out = jnp.sum(us * w_us + them * w_them, axis=1) + ob_ref[0, 0]
    out_ref[:, 0] = out


def kernel(white_features, black_features, stm, ft_weight, ft_bias, out_weight, out_bias):
    batch = white_features.shape[0]
    grid = (batch // BB,)
    out = pl.pallas_call(
        _tc_body,
        grid=grid,
        in_specs=[
            pl.BlockSpec((BB, MAX_ACTIVE), lambda i: (i, 0)),
            pl.BlockSpec((BB, MAX_ACTIVE), lambda i: (i, 0)),
            pl.BlockSpec((BB, 1), lambda i: (i, 0)),
            pl.BlockSpec((NUM_FEATURES, HIDDEN), lambda i: (0, 0)),
            pl.BlockSpec((1, HIDDEN), lambda i: (0, 0)),
            pl.BlockSpec((1, 2 * HIDDEN), lambda i: (0, 0)),
            pl.BlockSpec((1, 1), lambda i: (0, 0)),
        ],
        out_specs=pl.BlockSpec((BB, 1), lambda i: (i, 0)),
        out_shape=jax.ShapeDtypeStruct((batch, 1), jnp.float32),
    )(
        white_features,
        black_features,
        stm[:, None],
        ft_weight,
        ft_bias[None, :],
        out_weight[None, :],
        out_bias[None, :],
    )
    return out[:, 0]


# speedup vs baseline: 13.3484x; 1.0683x over previous
import functools

import jax
import jax.numpy as jnp
from jax import lax
from jax.experimental import pallas as pl
from jax.experimental.pallas import tpu as pltpu
from jax.experimental.pallas import tpu_sc as plsc

NUM_FEATURES = 768
HIDDEN = 512
MAX_ACTIVE = 32
BATCH = 4096

NUM_TILES = 32
ROWS_PER_TILE = BATCH // NUM_TILES
CHUNK_ROWS = 64
NUM_CHUNKS = ROWS_PER_TILE // CHUNK_ROWS
WIDTH = 2 * NUM_FEATURES

BB = 512


def _sc_counts_body(wf_hbm, bf_hbm, counts_hbm, featw_v, featb_v, counts_v):
    wid = lax.axis_index("s") * 2 + lax.axis_index("c")
    base = wid * ROWS_PER_TILE
    slab = ROWS_PER_TILE * MAX_ACTIVE
    pltpu.sync_copy(wf_hbm.at[pl.ds(base * MAX_ACTIVE, slab)], featw_v)
    pltpu.sync_copy(bf_hbm.at[pl.ds(base * MAX_ACTIVE, slab)], featb_v)

    lane = lax.iota(jnp.int32, 16)
    ones = jnp.ones((16,), jnp.float32)
    zeros = jnp.zeros((16,), jnp.float32)

    for chunk in range(NUM_CHUNKS):
        def zero_row(r, carry):
            for c in range(WIDTH // 16):
                counts_v[pl.ds(r * WIDTH + c * 16, 16)] = zeros
            return carry
        lax.fori_loop(0, CHUNK_ROWS, zero_row, 0)

        def scatter_group(g, carry):
            crow = g * 16 + lane
            frow = chunk * CHUNK_ROWS + crow
            cbase = crow * WIDTH
            fbase = frow * MAX_ACTIVE
            for a in range(MAX_ACTIVE):
                fw = plsc.load_gather(featw_v, [fbase + a])
                plsc.addupdate_scatter(counts_v, [cbase + fw], ones)
                fb = plsc.load_gather(featb_v, [fbase + a])
                plsc.addupdate_scatter(counts_v, [cbase + fb + NUM_FEATURES], ones)
            return carry
        lax.fori_loop(0, CHUNK_ROWS // 16, scatter_group, 0)

        pltpu.sync_copy(
            counts_v,
            counts_hbm.at[pl.ds((base + chunk * CHUNK_ROWS) * WIDTH,
                                CHUNK_ROWS * WIDTH)])


def _sc_counts(white_features, black_features):
    mesh = plsc.VectorSubcoreMesh(core_axis_name="c", subcore_axis_name="s")
    k = pl.kernel(
        _sc_counts_body,
        out_type=jax.ShapeDtypeStruct((BATCH * WIDTH,), jnp.float32),
        mesh=mesh,
        compiler_params=pltpu.CompilerParams(needs_layout_passes=False),
        scratch_types=[
            pltpu.VMEM((ROWS_PER_TILE * MAX_ACTIVE,), jnp.int32),
            pltpu.VMEM((ROWS_PER_TILE * MAX_ACTIVE,), jnp.int32),
            pltpu.VMEM((CHUNK_ROWS * WIDTH,), jnp.float32),
        ],
    )
    flat = k(white_features.reshape(-1), black_features.reshape(-1))
    return flat.reshape(BATCH, WIDTH)


def _tc_dense_body(counts_ref, stm_ref, table_ref, bias_ref, ow_ref, ob_ref, out_ref):
    c = counts_ref[...]
    table = table_ref[...]
    bias = bias_ref[0, :][None, :]
    acc_w = jnp.dot(c[:, :NUM_FEATURES], table,
                    preferred_element_type=jnp.float32) + bias
    acc_b = jnp.dot(c[:, NUM_FEATURES:], table,
                    preferred_element_type=jnp.float32) + bias

    act_w = jnp.square(jnp.clip(acc_w, 0.0, 1.0))
    act_b = jnp.square(jnp.clip(acc_b, 0.0, 1.0))

    s = stm_ref[:, 0].astype(jnp.float32)[:, None]
    us = act_w + s * (act_b - act_w)
    them = act_b + s * (act_w - act_b)

    w_us = ow_ref[0, :HIDDEN][None, :]
    w_them = ow_ref[0, HIDDEN:][None, :]
    out = jnp.sum(us * w_us + them * w_them, axis=1) + ob_ref[0, 0]
    out_ref[:, 0] = out


def _tc_dense(counts, stm, ft_weight, ft_bias, out_weight, out_bias):
    grid = (BATCH // BB,)
    out = pl.pallas_call(
        _tc_dense_body,
        grid=grid,
        in_specs=[
            pl.BlockSpec((BB, WIDTH), lambda i: (i, 0)),
            pl.BlockSpec((BB, 1), lambda i: (i, 0)),
            pl.BlockSpec((NUM_FEATURES, HIDDEN), lambda i: (0, 0)),
            pl.BlockSpec((1, HIDDEN), lambda i: (0, 0)),
            pl.BlockSpec((1, 2 * HIDDEN), lambda i: (0, 0)),
            pl.BlockSpec((1, 1), lambda i: (0, 0)),
        ],
        out_specs=pl.BlockSpec((BB, 1), lambda i: (i, 0)),
        out_shape=jax.ShapeDtypeStruct((BATCH, 1), jnp.float32),
    )(
        counts,
        stm[:, None],
        ft_weight,
        ft_bias[None, :],
        out_weight[None, :],
        out_bias[None, :],
    )
    return out[:, 0]


def kernel(white_features, black_features, stm, ft_weight, ft_bias, out_weight, out_bias):
    counts = _sc_counts(white_features, black_features)
    return _tc_dense(counts, stm, ft_weight, ft_bias, out_weight, out_bias)


# speedup vs baseline: 17.9918x; 1.3479x over previous
import functools

import jax
import jax.numpy as jnp
from jax import lax
from jax.experimental import pallas as pl
from jax.experimental.pallas import tpu as pltpu
from jax.experimental.pallas import tpu_sc as plsc

NUM_FEATURES = 768
HIDDEN = 512
MAX_ACTIVE = 32
BATCH = 4096

NUM_TILES = 32
ROWS_PER_TILE = BATCH // NUM_TILES
CHUNK_ROWS = 64
NUM_CHUNKS = ROWS_PER_TILE // CHUNK_ROWS
WIDTH = 2 * NUM_FEATURES

BB = 512


def _sc_counts_body(wf_hbm, bf_hbm, counts_hbm, featw_v, featb_v, counts_v):
    wid = lax.axis_index("s") * 2 + lax.axis_index("c")
    base = wid * ROWS_PER_TILE
    pltpu.sync_copy(wf_hbm.at[pl.ds(base, ROWS_PER_TILE), :], featw_v)
    pltpu.sync_copy(bf_hbm.at[pl.ds(base, ROWS_PER_TILE), :], featb_v)

    lane = lax.iota(jnp.int32, 16)
    ones = jnp.ones((16,), jnp.float32)
    zeros = jnp.zeros((16,), jnp.float32)

    for chunk in range(NUM_CHUNKS):
        def zero_row(r, carry):
            for c in range(WIDTH // 16):
                counts_v[r, pl.ds(c * 16, 16)] = zeros
            return carry
        lax.fori_loop(0, CHUNK_ROWS, zero_row, 0)

        def scatter_group(g, carry):
            crow = g * 16 + lane
            frow = chunk * CHUNK_ROWS + crow
            for a in range(MAX_ACTIVE):
                col = jnp.full((16,), a, jnp.int32)
                fw = plsc.load_gather(featw_v, [frow, col])
                plsc.addupdate_scatter(counts_v, [crow, fw], ones)
                fb = plsc.load_gather(featb_v, [frow, col])
                plsc.addupdate_scatter(counts_v, [crow, fb + NUM_FEATURES], ones)
            return carry
        lax.fori_loop(0, CHUNK_ROWS // 16, scatter_group, 0)

        pltpu.sync_copy(
            counts_v,
            counts_hbm.at[pl.ds(base + chunk * CHUNK_ROWS, CHUNK_ROWS), :])


def _sc_counts(white_features, black_features):
    mesh = plsc.VectorSubcoreMesh(core_axis_name="c", subcore_axis_name="s")
    k = pl.kernel(
        _sc_counts_body,
        out_type=jax.ShapeDtypeStruct((BATCH, WIDTH), jnp.float32),
        mesh=mesh,
        compiler_params=pltpu.CompilerParams(needs_layout_passes=False),
        scratch_types=[
            pltpu.VMEM((ROWS_PER_TILE, MAX_ACTIVE), jnp.int32),
            pltpu.VMEM((ROWS_PER_TILE, MAX_ACTIVE), jnp.int32),
            pltpu.VMEM((CHUNK_ROWS, WIDTH), jnp.float32),
        ],
    )
    return k(white_features, black_features)


def _tc_dense_body(counts_ref, stm_ref, table_ref, bias_ref, ow_ref, ob_ref, out_ref):
    c = counts_ref[...]
    table = table_ref[...]
    bias = bias_ref[0, :][None, :]
    acc_w = jnp.dot(c[:, :NUM_FEATURES], table,
                    preferred_element_type=jnp.float32) + bias
    acc_b = jnp.dot(c[:, NUM_FEATURES:], table,
                    preferred_element_type=jnp.float32) + bias

    act_w = jnp.square(jnp.clip(acc_w, 0.0, 1.0))
    act_b = jnp.square(jnp.clip(acc_b, 0.0, 1.0))

    s = stm_ref[:, 0].astype(jnp.float32)[:, None]
    us = act_w + s * (act_b - act_w)
    them = act_b + s * (act_w - act_b)

    w_us = ow_ref[0, :HIDDEN][None, :]
    w_them = ow_ref[0, HIDDEN:][None, :]
    out = jnp.sum(us * w_us + them * w_them, axis=1) + ob_ref[0, 0]
    out_ref[:, 0] = out


def _tc_dense(counts, stm, ft_weight, ft_bias, out_weight, out_bias):
    grid = (BATCH // BB,)
    out = pl.pallas_call(
        _tc_dense_body,
        grid=grid,
        in_specs=[
            pl.BlockSpec((BB, WIDTH), lambda i: (i, 0)),
            pl.BlockSpec((BB, 1), lambda i: (i, 0)),
            pl.BlockSpec((NUM_FEATURES, HIDDEN), lambda i: (0, 0)),
            pl.BlockSpec((1, HIDDEN), lambda i: (0, 0)),
            pl.BlockSpec((1, 2 * HIDDEN), lambda i: (0, 0)),
            pl.BlockSpec((1, 1), lambda i: (0, 0)),
        ],
        out_specs=pl.BlockSpec((BB, 1), lambda i: (i, 0)),
        out_shape=jax.ShapeDtypeStruct((BATCH, 1), jnp.float32),
    )(
        counts,
        stm[:, None],
        ft_weight,
        ft_bias[None, :],
        out_weight[None, :],
        out_bias[None, :],
    )
    return out[:, 0]


def kernel(white_features, black_features, stm, ft_weight, ft_bias, out_weight, out_bias):
    counts = _sc_counts(white_features, black_features)
    return _tc_dense(counts, stm, ft_weight, ft_bias, out_weight, out_bias)


# speedup vs baseline: 18.9197x; 1.0516x over previous
import functools

import jax
import jax.numpy as jnp
from jax import lax
from jax.experimental import pallas as pl
from jax.experimental.pallas import tpu as pltpu
from jax.experimental.pallas import tpu_sc as plsc

NUM_FEATURES = 768
HIDDEN = 512
MAX_ACTIVE = 32
BATCH = 4096

NUM_TILES = 32
ROWS_PER_TILE = BATCH // NUM_TILES
CHUNK_ROWS = 32
NUM_CHUNKS = ROWS_PER_TILE // CHUNK_ROWS
WIDTH = 2 * NUM_FEATURES

BB = 512


def _sc_counts_body(wf_hbm, bf_hbm, counts_hbm,
                    featw_v, featb_v, counts_a, counts_b, sem_a, sem_b):
    wid = lax.axis_index("s") * 2 + lax.axis_index("c")
    base = wid * ROWS_PER_TILE
    pltpu.sync_copy(wf_hbm.at[pl.ds(base, ROWS_PER_TILE), :], featw_v)
    pltpu.sync_copy(bf_hbm.at[pl.ds(base, ROWS_PER_TILE), :], featb_v)

    lane = lax.iota(jnp.int32, 16)
    ones = jnp.ones((16,), jnp.float32)
    zeros = jnp.zeros((16,), jnp.float32)

    bufs = (counts_a, counts_b)
    sems = (sem_a, sem_b)
    copies = [None] * NUM_CHUNKS
    for chunk in range(NUM_CHUNKS):
        counts_v = bufs[chunk % 2]
        if chunk >= 2:
            copies[chunk - 2].wait()

        def zero_row(r, carry):
            for c in range(WIDTH // 16):
                counts_v[r, pl.ds(c * 16, 16)] = zeros
            return carry
        lax.fori_loop(0, CHUNK_ROWS, zero_row, 0)

        def scatter_group(g, carry):
            crow = g * 16 + lane
            frow = chunk * CHUNK_ROWS + crow
            for a in range(MAX_ACTIVE):
                col = jnp.full((16,), a, jnp.int32)
                fw = plsc.load_gather(featw_v, [frow, col])
                plsc.addupdate_scatter(counts_v, [crow, fw], ones)
                fb = plsc.load_gather(featb_v, [frow, col])
                plsc.addupdate_scatter(counts_v, [crow, fb + NUM_FEATURES], ones)
            return carry
        lax.fori_loop(0, CHUNK_ROWS // 16, scatter_group, 0)

        copies[chunk] = pltpu.make_async_copy(
            counts_v,
            counts_hbm.at[pl.ds(base + chunk * CHUNK_ROWS, CHUNK_ROWS), :],
            sems[chunk % 2])
        copies[chunk].start()
    for chunk in range(max(0, NUM_CHUNKS - 2), NUM_CHUNKS):
        copies[chunk].wait()


def _sc_counts(white_features, black_features):
    mesh = plsc.VectorSubcoreMesh(core_axis_name="c", subcore_axis_name="s")
    k = pl.kernel(
        _sc_counts_body,
        out_type=jax.ShapeDtypeStruct((BATCH, WIDTH), jnp.float32),
        mesh=mesh,
        compiler_params=pltpu.CompilerParams(needs_layout_passes=False),
        scratch_types=[
            pltpu.VMEM((ROWS_PER_TILE, MAX_ACTIVE), jnp.int32),
            pltpu.VMEM((ROWS_PER_TILE, MAX_ACTIVE), jnp.int32),
            pltpu.VMEM((CHUNK_ROWS, WIDTH), jnp.float32),
            pltpu.VMEM((CHUNK_ROWS, WIDTH), jnp.float32),
            pltpu.SemaphoreType.DMA,
            pltpu.SemaphoreType.DMA,
        ],
    )
    return k(white_features, black_features)


def _tc_dense_body(counts_ref, stm_ref, table_ref, bias_ref, ow_ref, ob_ref, out_ref):
    c = counts_ref[...]
    table = table_ref[...]
    bias = bias_ref[0, :][None, :]
    acc_w = jnp.dot(c[:, :NUM_FEATURES], table,
                    preferred_element_type=jnp.float32) + bias
    acc_b = jnp.dot(c[:, NUM_FEATURES:], table,
                    preferred_element_type=jnp.float32) + bias

    act_w = jnp.square(jnp.clip(acc_w, 0.0, 1.0))
    act_b = jnp.square(jnp.clip(acc_b, 0.0, 1.0))

    s = stm_ref[:, 0].astype(jnp.float32)[:, None]
    us = act_w + s * (act_b - act_w)
    them = act_b + s * (act_w - act_b)

    w_us = ow_ref[0, :HIDDEN][None, :]
    w_them = ow_ref[0, HIDDEN:][None, :]
    out = jnp.sum(us * w_us + them * w_them, axis=1) + ob_ref[0, 0]
    out_ref[:, 0] = out


def _tc_dense(counts, stm, ft_weight, ft_bias, out_weight, out_bias):
    grid = (BATCH // BB,)
    out = pl.pallas_call(
        _tc_dense_body,
        grid=grid,
        in_specs=[
            pl.BlockSpec((BB, WIDTH), lambda i: (i, 0)),
            pl.BlockSpec((BB, 1), lambda i: (i, 0)),
            pl.BlockSpec((NUM_FEATURES, HIDDEN), lambda i: (0, 0)),
            pl.BlockSpec((1, HIDDEN), lambda i: (0, 0)),
            pl.BlockSpec((1, 2 * HIDDEN), lambda i: (0, 0)),
            pl.BlockSpec((1, 1), lambda i: (0, 0)),
        ],
        out_specs=pl.BlockSpec((BB, 1), lambda i: (i, 0)),
        out_shape=jax.ShapeDtypeStruct((BATCH, 1), jnp.float32),
    )(
        counts,
        stm[:, None],
        ft_weight,
        ft_bias[None, :],
        out_weight[None, :],
        out_bias[None, :],
    )
    return out[:, 0]


def kernel(white_features, black_features, stm, ft_weight, ft_bias, out_weight, out_bias):
    counts = _sc_counts(white_features, black_features)
    return _tc_dense(counts, stm, ft_weight, ft_bias, out_weight, out_bias)


# speedup vs baseline: 22.5955x; 1.1943x over previous
import functools

import jax
import jax.numpy as jnp
from jax import lax
from jax.experimental import pallas as pl
from jax.experimental.pallas import tpu as pltpu
from jax.experimental.pallas import tpu_sc as plsc

NUM_FEATURES = 768
HIDDEN = 512
MAX_ACTIVE = 32
BATCH = 4096

NUM_TILES = 32
ROWS_PER_TILE = BATCH // NUM_TILES
CHUNK_ROWS = 64
NUM_CHUNKS = ROWS_PER_TILE // CHUNK_ROWS
WORDS = 512

BB = 512


def _sc_counts_body(wf_hbm, bf_hbm, counts_hbm,
                    featw_v, featb_v, counts_a, counts_b, sem_a, sem_b):
    wid = lax.axis_index("s") * 2 + lax.axis_index("c")
    base = wid * ROWS_PER_TILE
    pltpu.sync_copy(wf_hbm.at[pl.ds(base, ROWS_PER_TILE), :], featw_v)
    pltpu.sync_copy(bf_hbm.at[pl.ds(base, ROWS_PER_TILE), :], featb_v)

    lane = lax.iota(jnp.int32, 16)
    izeros = jnp.zeros((16,), jnp.int32)
    ones = jnp.ones((16,), jnp.int32)
    eights = jnp.full((16,), 8, jnp.int32)

    bufs = (counts_a, counts_b)
    sems = (sem_a, sem_b)
    copies = [None] * NUM_CHUNKS
    for chunk in range(NUM_CHUNKS):
        counts_v = bufs[chunk % 2]
        if chunk >= 2:
            copies[chunk - 2].wait()

        def zero_row(r, carry):
            for c in range(WORDS // 16):
                counts_v[r, pl.ds(c * 16, 16)] = izeros
            return carry
        lax.fori_loop(0, CHUNK_ROWS, zero_row, 0)

        def scatter_group(g, carry):
            crow = g * 16 + lane
            frow = chunk * CHUNK_ROWS + crow
            for a in range(MAX_ACTIVE):
                col = jnp.full((16,), a, jnp.int32)
                fw = plsc.load_gather(featw_v, [frow, col])
                val_w = ones << ((fw >> 9) * eights)
                plsc.addupdate_scatter(counts_v, [crow, fw & (WORDS - 1)], val_w)
                gb = plsc.load_gather(featb_v, [frow, col]) + 1024
                val_b = ones << ((gb >> 9) * eights)
                plsc.addupdate_scatter(counts_v, [crow, gb & (WORDS - 1)], val_b)
            return carry
        lax.fori_loop(0, CHUNK_ROWS // 16, scatter_group, 0)

        copies[chunk] = pltpu.make_async_copy(
            counts_v,
            counts_hbm.at[pl.ds(base + chunk * CHUNK_ROWS, CHUNK_ROWS), :],
            sems[chunk % 2])
        copies[chunk].start()
    for chunk in range(max(0, NUM_CHUNKS - 2), NUM_CHUNKS):
        copies[chunk].wait()


def _sc_counts(white_features, black_features):
    mesh = plsc.VectorSubcoreMesh(core_axis_name="c", subcore_axis_name="s")
    k = pl.kernel(
        _sc_counts_body,
        out_type=jax.ShapeDtypeStruct((BATCH, WORDS), jnp.int32),
        mesh=mesh,
        compiler_params=pltpu.CompilerParams(needs_layout_passes=False),
        scratch_types=[
            pltpu.VMEM((ROWS_PER_TILE, MAX_ACTIVE), jnp.int32),
            pltpu.VMEM((ROWS_PER_TILE, MAX_ACTIVE), jnp.int32),
            pltpu.VMEM((CHUNK_ROWS, WORDS), jnp.int32),
            pltpu.VMEM((CHUNK_ROWS, WORDS), jnp.int32),
            pltpu.SemaphoreType.DMA,
            pltpu.SemaphoreType.DMA,
        ],
    )
    return k(white_features, black_features)


def _tc_dense_body(counts_ref, stm_ref, table_ref, bias_ref, ow_ref, ob_ref, out_ref):
    w = counts_ref[...]
    t_lo = table_ref[:WORDS, :]
    t_hi = table_ref[WORDS:, :]
    bias = bias_ref[0, :][None, :]

    def acc_of(p_lo, p_hi):
        f_lo = (p_lo & 255).astype(jnp.float32)
        f_hi = (p_hi & 255).astype(jnp.float32)
        return (jnp.dot(f_lo, t_lo, preferred_element_type=jnp.float32)
                + jnp.dot(f_hi[:, :NUM_FEATURES - WORDS], t_hi,
                          preferred_element_type=jnp.float32) + bias)

    acc_w = acc_of(w, w >> 8)
    acc_b = acc_of(w >> 16, w >> 24)

    act_w = jnp.square(jnp.clip(acc_w, 0.0, 1.0))
    act_b = jnp.square(jnp.clip(acc_b, 0.0, 1.0))

    s = stm_ref[:, 0].astype(jnp.float32)[:, None]
    us = act_w + s * (act_b - act_w)
    them = act_b + s * (act_w - act_b)

    w_us = ow_ref[0, :HIDDEN][None, :]
    w_them = ow_ref[0, HIDDEN:][None, :]
    out_ref[...] = jnp.sum(us * w_us + them * w_them, axis=1) + ob_ref[0, 0]


def _tc_dense(counts, stm, ft_weight, ft_bias, out_weight, out_bias):
    grid = (BATCH // BB,)
    return pl.pallas_call(
        _tc_dense_body,
        grid=grid,
        in_specs=[
            pl.BlockSpec((BB, WORDS), lambda i: (i, 0)),
            pl.BlockSpec((BB, 1), lambda i: (i, 0)),
            pl.BlockSpec((NUM_FEATURES, HIDDEN), lambda i: (0, 0)),
            pl.BlockSpec((1, HIDDEN), lambda i: (0, 0)),
            pl.BlockSpec((1, 2 * HIDDEN), lambda i: (0, 0)),
            pl.BlockSpec((1, 1), lambda i: (0, 0)),
        ],
        out_specs=pl.BlockSpec((BB,), lambda i: (i,)),
        out_shape=jax.ShapeDtypeStruct((BATCH,), jnp.float32),
    )(
        counts,
        stm[:, None],
        ft_weight,
        ft_bias[None, :],
        out_weight[None, :],
        out_bias[None, :],
    )


def kernel(white_features, black_features, stm, ft_weight, ft_bias, out_weight, out_bias):
    counts = _sc_counts(white_features, black_features)
    return _tc_dense(counts, stm, ft_weight, ft_bias, out_weight, out_bias)


# speedup vs baseline: 24.4867x; 1.0837x over previous
import functools

import jax
import jax.numpy as jnp
from jax import lax
from jax.experimental import pallas as pl
from jax.experimental.pallas import tpu as pltpu
from jax.experimental.pallas import tpu_sc as plsc

NUM_FEATURES = 768
HIDDEN = 512
MAX_ACTIVE = 32
BATCH = 4096

NUM_TILES = 32
ROWS_PER_TILE = BATCH // NUM_TILES
CHUNK_ROWS = 64
NUM_CHUNKS = ROWS_PER_TILE // CHUNK_ROWS
WORDS = 512

BB = 512


def _sc_counts_body(wf_hbm, bf_hbm, counts_hbm,
                    featw_v, featb_v, counts_a, counts_b, sem_a, sem_b):
    wid = lax.axis_index("s") * 2 + lax.axis_index("c")
    base = wid * ROWS_PER_TILE
    pltpu.sync_copy(wf_hbm.at[:, pl.ds(base, ROWS_PER_TILE)], featw_v)
    pltpu.sync_copy(bf_hbm.at[:, pl.ds(base, ROWS_PER_TILE)], featb_v)

    lane = lax.iota(jnp.int32, 16)
    izeros = jnp.zeros((16,), jnp.int32)
    ones = jnp.ones((16,), jnp.int32)
    eights = jnp.full((16,), 8, jnp.int32)

    bufs = (counts_a, counts_b)
    sems = (sem_a, sem_b)
    copies = [None] * NUM_CHUNKS
    for chunk in range(NUM_CHUNKS):
        counts_v = bufs[chunk % 2]
        if chunk >= 2:
            copies[chunk - 2].wait()

        def zero_row(r, carry):
            for c in range(WORDS // 16):
                counts_v[r, pl.ds(c * 16, 16)] = izeros
            return carry
        lax.fori_loop(0, CHUNK_ROWS, zero_row, 0)

        def scatter_group(g, carry):
            crow = g * 16 + lane
            foff = chunk * CHUNK_ROWS + g * 16
            for a in range(MAX_ACTIVE):
                fw = featw_v[a, pl.ds(foff, 16)]
                val_w = ones << ((fw >> 9) * eights)
                plsc.addupdate_scatter(counts_v, [crow, fw & (WORDS - 1)], val_w)
                gb = featb_v[a, pl.ds(foff, 16)] + 1024
                val_b = ones << ((gb >> 9) * eights)
                plsc.addupdate_scatter(counts_v, [crow, gb & (WORDS - 1)], val_b)
            return carry
        lax.fori_loop(0, CHUNK_ROWS // 16, scatter_group, 0)

        copies[chunk] = pltpu.make_async_copy(
            counts_v,
            counts_hbm.at[pl.ds(base + chunk * CHUNK_ROWS, CHUNK_ROWS), :],
            sems[chunk % 2])
        copies[chunk].start()
    for chunk in range(max(0, NUM_CHUNKS - 2), NUM_CHUNKS):
        copies[chunk].wait()


def _sc_counts(white_features, black_features):
    mesh = plsc.VectorSubcoreMesh(core_axis_name="c", subcore_axis_name="s")
    k = pl.kernel(
        _sc_counts_body,
        out_type=jax.ShapeDtypeStruct((BATCH, WORDS), jnp.int32),
        mesh=mesh,
        compiler_params=pltpu.CompilerParams(needs_layout_passes=False),
        scratch_types=[
            pltpu.VMEM((MAX_ACTIVE, ROWS_PER_TILE), jnp.int32),
            pltpu.VMEM((MAX_ACTIVE, ROWS_PER_TILE), jnp.int32),
            pltpu.VMEM((CHUNK_ROWS, WORDS), jnp.int32),
            pltpu.VMEM((CHUNK_ROWS, WORDS), jnp.int32),
            pltpu.SemaphoreType.DMA,
            pltpu.SemaphoreType.DMA,
        ],
    )
    return k(white_features.T, black_features.T)


def _tc_dense_body(counts_ref, stm_ref, table_ref, bias_ref, ow_ref, ob_ref, out_ref):
    w = counts_ref[...]
    t_lo = table_ref[:WORDS, :]
    t_hi = table_ref[WORDS:, :]
    bias = bias_ref[0, :][None, :]

    def acc_of(p_lo, p_hi):
        f_lo = (p_lo & 255).astype(jnp.float32)
        f_hi = (p_hi & 255).astype(jnp.float32)
        return (jnp.dot(f_lo, t_lo, preferred_element_type=jnp.float32)
                + jnp.dot(f_hi[:, :NUM_FEATURES - WORDS], t_hi,
                          preferred_element_type=jnp.float32) + bias)

    acc_w = acc_of(w, w >> 8)
    acc_b = acc_of(w >> 16, w >> 24)

    act_w = jnp.square(jnp.clip(acc_w, 0.0, 1.0))
    act_b = jnp.square(jnp.clip(acc_b, 0.0, 1.0))

    w_us = ow_ref[0, :HIDDEN][None, :]
    w_them = ow_ref[0, HIDDEN:][None, :]
    p_w_us = jnp.sum(act_w * w_us, axis=1)
    p_w_them = jnp.sum(act_w * w_them, axis=1)
    p_b_us = jnp.sum(act_b * w_us, axis=1)
    p_b_them = jnp.sum(act_b * w_them, axis=1)
    s = stm_ref[...].astype(jnp.float32)
    out_ref[...] = (p_w_us + p_b_them
                    + s * (p_b_us + p_w_them - p_w_us - p_b_them)
                    + ob_ref[0, 0])


def _tc_dense(counts, stm, ft_weight, ft_bias, out_weight, out_bias):
    grid = (BATCH // BB,)
    return pl.pallas_call(
        _tc_dense_body,
        grid=grid,
        in_specs=[
            pl.BlockSpec((BB, WORDS), lambda i: (i, 0)),
            pl.BlockSpec((BB,), lambda i: (i,)),
            pl.BlockSpec((NUM_FEATURES, HIDDEN), lambda i: (0, 0)),
            pl.BlockSpec((1, HIDDEN), lambda i: (0, 0)),
            pl.BlockSpec((1, 2 * HIDDEN), lambda i: (0, 0)),
            pl.BlockSpec((1, 1), lambda i: (0, 0)),
        ],
        out_specs=pl.BlockSpec((BB,), lambda i: (i,)),
        out_shape=jax.ShapeDtypeStruct((BATCH,), jnp.float32),
    )(
        counts,
        stm,
        ft_weight,
        ft_bias[None, :],
        out_weight[None, :],
        out_bias[None, :],
    )


def kernel(white_features, black_features, stm, ft_weight, ft_bias, out_weight, out_bias):
    counts = _sc_counts(white_features, black_features)
    return _tc_dense(counts, stm, ft_weight, ft_bias, out_weight, out_bias)
